# initial kernel scaffold (unmeasured)
import jax
import jax.numpy as jnp
from jax import lax
from jax.experimental import pallas as pl
from jax.experimental.pallas import tpu as pltpu

N_DEV = 4
M_PER = 2048
K = 2048
F_LOC = 8192
F_T = 512
N_FT = F_LOC // F_T


def kernel(x, W1, W2):
    def body(x_ref, w1_ref, w2_ref, out_ref,
             x_all, p_all, rs_recv,
             x_vm, acc_vm, w1_vm, w2_vm,
             ag_send_sems, ag_recv_sems, rs_send_sems, rs_recv_sems,
             local_sems):
        my = lax.axis_index("i")
        right = lax.rem(my + 1, N_DEV)
        left = lax.rem(my + N_DEV - 1, N_DEV)

        barrier = pltpu.get_barrier_semaphore()
        for nbr in (left, right):
            pl.semaphore_signal(barrier, inc=1, device_id=(nbr,),
                                device_id_type=pl.DeviceIdType.MESH)
        pl.semaphore_wait(barrier, 2)

        for h in range(N_DEV - 1):
            src = x_ref if h == 0 else x_all.at[h]
            rdma = pltpu.make_async_remote_copy(
                src_ref=src,
                dst_ref=x_all.at[h + 1],
                send_sem=ag_send_sems.at[h],
                recv_sem=ag_recv_sems.at[h],
                device_id=(right,),
                device_id_type=pl.DeviceIdType.MESH,
            )
            rdma.start()
            rdma.wait()

        def compute_chunk(k):
            src = x_ref if k == 0 else x_all.at[k]
            cp = pltpu.make_async_copy(src, x_vm, local_sems.at[2])
            cp.start()
            cp.wait()
            acc_vm[...] = jnp.zeros_like(acc_vm)

            def t_body(t, carry):
                c1 = pltpu.make_async_copy(
                    w1_ref.at[:, pl.ds(t * F_T, F_T)], w1_vm, local_sems.at[0])
                c2 = pltpu.make_async_copy(
                    w2_ref.at[pl.ds(t * F_T, F_T), :], w2_vm, local_sems.at[1])
                c1.start()
                c2.start()
                c1.wait()
                c2.wait()
                h = jnp.dot(x_vm[...].astype(jnp.bfloat16),
                            w1_vm[...].astype(jnp.bfloat16),
                            preferred_element_type=jnp.float32)
                s = h * jax.nn.sigmoid(h)
                acc_vm[...] += jnp.dot(s.astype(jnp.bfloat16),
                                       w2_vm[...].astype(jnp.bfloat16),
                                       preferred_element_type=jnp.float32)
                return carry

            lax.fori_loop(0, N_FT, t_body, 0)
            st = pltpu.make_async_copy(acc_vm, p_all.at[k], local_sems.at[3])
            st.start()
            st.wait()

        for k in range(N_DEV):
            compute_chunk(k)

        for s in range(N_DEV - 1):
            src = p_all.at[1] if s == 0 else x_vm
            rdma = pltpu.make_async_remote_copy(
                src_ref=src,
                dst_ref=rs_recv.at[s],
                send_sem=rs_send_sems.at[s],
                recv_sem=rs_recv_sems.at[s],
                device_id=(right,),
                device_id_type=pl.DeviceIdType.MESH,
            )
            rdma.start()
            rdma.wait()
            ld1 = pltpu.make_async_copy(rs_recv.at[s], x_vm, local_sems.at[2])
            ld2 = pltpu.make_async_copy(p_all.at[(s + 2) % N_DEV], acc_vm,
                                        local_sems.at[3])
            ld1.start()
            ld2.start()
            ld1.wait()
            ld2.wait()
            x_vm[...] = x_vm[...] + acc_vm[...]

        st = pltpu.make_async_copy(x_vm, out_ref, local_sems.at[3])
        st.start()
        st.wait()

    return pl.pallas_call(
        body,
        out_shape=jax.ShapeDtypeStruct((M_PER, K), jnp.float32),
        in_specs=[
            pl.BlockSpec(memory_space=pl.ANY),
            pl.BlockSpec(memory_space=pl.ANY),
            pl.BlockSpec(memory_space=pl.ANY),
        ],
        out_specs=pl.BlockSpec(memory_space=pl.ANY),
        scratch_shapes=[
            pltpu.MemorySpace.HBM((N_DEV, M_PER, K), jnp.float32),
            pltpu.MemorySpace.HBM((N_DEV, M_PER, K), jnp.float32),
            pltpu.MemorySpace.HBM((N_DEV - 1, M_PER, K), jnp.float32),
            pltpu.VMEM((M_PER, K), jnp.float32),
            pltpu.VMEM((M_PER, K), jnp.float32),
            pltpu.VMEM((K, F_T), jnp.float32),
            pltpu.VMEM((F_T, K), jnp.float32),
            pltpu.SemaphoreType.DMA((N_DEV - 1,)),
            pltpu.SemaphoreType.DMA((N_DEV - 1,)),
            pltpu.SemaphoreType.DMA((N_DEV - 1,)),
            pltpu.SemaphoreType.DMA((N_DEV - 1,)),
            pltpu.SemaphoreType.DMA((4,)),
        ],
        compiler_params=pltpu.CompilerParams(collective_id=0),
    )(x, W1, W2)


# baseline (device time: 2037640 ns/iter reference)
import jax
import jax.numpy as jnp
from jax import lax
from jax.experimental import pallas as pl
from jax.experimental.pallas import tpu as pltpu

N_DEV = 4
M_PER = 2048
K = 2048
F_LOC = 8192
F_T = 512
N_FT = F_LOC // F_T


def kernel(x, W1, W2):
    def body(x_ref, w1_ref, w2_ref,
             out_ref, x_all, p_all, rs_recv,
             x_vm, acc_vm, w1_vm, w2_vm,
             ag_send_sems, ag_recv_sems, rs_send_sems, rs_recv_sems,
             local_sems):
        my = lax.axis_index("i")
        right = lax.rem(my + 1, N_DEV)
        left = lax.rem(my + N_DEV - 1, N_DEV)

        barrier = pltpu.get_barrier_semaphore()
        for nbr in (left, right):
            pl.semaphore_signal(barrier, inc=1, device_id=(nbr,),
                                device_id_type=pl.DeviceIdType.MESH)
        pl.semaphore_wait(barrier, 2)

        for h in range(N_DEV - 1):
            src = x_ref if h == 0 else x_all.at[h]
            rdma = pltpu.make_async_remote_copy(
                src_ref=src,
                dst_ref=x_all.at[h + 1],
                send_sem=ag_send_sems.at[h],
                recv_sem=ag_recv_sems.at[h],
                device_id=(right,),
                device_id_type=pl.DeviceIdType.MESH,
            )
            rdma.start()
            rdma.wait()

        def compute_chunk(k):
            src = x_ref if k == 0 else x_all.at[k]
            cp = pltpu.make_async_copy(src, x_vm, local_sems.at[2])
            cp.start()
            cp.wait()
            acc_vm[...] = jnp.zeros_like(acc_vm)

            def t_body(t, carry):
                c1 = pltpu.make_async_copy(
                    w1_ref.at[:, pl.ds(t * F_T, F_T)], w1_vm, local_sems.at[0])
                c2 = pltpu.make_async_copy(
                    w2_ref.at[pl.ds(t * F_T, F_T), :], w2_vm, local_sems.at[1])
                c1.start()
                c2.start()
                c1.wait()
                c2.wait()
                h = jnp.dot(x_vm[...].astype(jnp.bfloat16),
                            w1_vm[...].astype(jnp.bfloat16),
                            preferred_element_type=jnp.float32)
                s = h * jax.nn.sigmoid(h)
                acc_vm[...] += jnp.dot(s.astype(jnp.bfloat16),
                                       w2_vm[...].astype(jnp.bfloat16),
                                       preferred_element_type=jnp.float32)
                return carry

            lax.fori_loop(0, N_FT, t_body, 0)
            st = pltpu.make_async_copy(acc_vm, p_all.at[k], local_sems.at[3])
            st.start()
            st.wait()

        for k in range(N_DEV):
            compute_chunk(k)

        for s in range(N_DEV - 1):
            src = p_all.at[1] if s == 0 else x_vm
            rdma = pltpu.make_async_remote_copy(
                src_ref=src,
                dst_ref=rs_recv.at[s],
                send_sem=rs_send_sems.at[s],
                recv_sem=rs_recv_sems.at[s],
                device_id=(right,),
                device_id_type=pl.DeviceIdType.MESH,
            )
            rdma.start()
            rdma.wait()
            ld1 = pltpu.make_async_copy(rs_recv.at[s], x_vm, local_sems.at[2])
            ld2 = pltpu.make_async_copy(p_all.at[(s + 2) % N_DEV], acc_vm,
                                        local_sems.at[3])
            ld1.start()
            ld2.start()
            ld1.wait()
            ld2.wait()
            x_vm[...] = x_vm[...] + acc_vm[...]

        st = pltpu.make_async_copy(x_vm, out_ref, local_sems.at[3])
        st.start()
        st.wait()

    out, _, _, _ = pl.pallas_call(
        body,
        out_shape=[
            jax.ShapeDtypeStruct((M_PER, K), jnp.float32),
            jax.ShapeDtypeStruct((N_DEV, M_PER, K), jnp.float32),
            jax.ShapeDtypeStruct((N_DEV, M_PER, K), jnp.float32),
            jax.ShapeDtypeStruct((N_DEV - 1, M_PER, K), jnp.float32),
        ],
        in_specs=[
            pl.BlockSpec(memory_space=pl.ANY),
            pl.BlockSpec(memory_space=pl.ANY),
            pl.BlockSpec(memory_space=pl.ANY),
        ],
        out_specs=[
            pl.BlockSpec(memory_space=pl.ANY),
            pl.BlockSpec(memory_space=pl.ANY),
            pl.BlockSpec(memory_space=pl.ANY),
            pl.BlockSpec(memory_space=pl.ANY),
        ],
        scratch_shapes=[
            pltpu.VMEM((M_PER, K), jnp.float32),
            pltpu.VMEM((M_PER, K), jnp.float32),
            pltpu.VMEM((K, F_T), jnp.float32),
            pltpu.VMEM((F_T, K), jnp.float32),
            pltpu.SemaphoreType.DMA((N_DEV - 1,)),
            pltpu.SemaphoreType.DMA((N_DEV - 1,)),
            pltpu.SemaphoreType.DMA((N_DEV - 1,)),
            pltpu.SemaphoreType.DMA((N_DEV - 1,)),
            pltpu.SemaphoreType.DMA((4,)),
        ],
        compiler_params=pltpu.CompilerParams(
            collective_id=0,
            vmem_limit_bytes=60 * 1024 * 1024,
        ),
    )(x, W1, W2)
    return out


# device time: 800368 ns/iter; 2.5459x vs baseline; 2.5459x over previous
import jax
import jax.numpy as jnp
from jax import lax
from jax.experimental import pallas as pl
from jax.experimental.pallas import tpu as pltpu

N_DEV = 4
M_PER = 2048
K = 2048
F_LOC = 8192
F_T = 512
N_FT = F_LOC // F_T


def kernel(x, W1, W2):
    def body(x_ref, w1_ref, w2_ref,
             out_ref, x_all, rs_recv, p0_buf,
             x_bf, acc_vm, w1_vm, w2_vm, rs_send_vm, rtmp_vm,
             ag_send_sems, ag_recv_sems, rs_send_sems, rs_recv_sems,
             w1_sems, w2_sems, local_sems):
        my = lax.axis_index("i")
        right = lax.rem(my + 1, N_DEV)
        left = lax.rem(my + N_DEV - 1, N_DEV)

        barrier = pltpu.get_barrier_semaphore()
        for nbr in (left, right):
            pl.semaphore_signal(barrier, inc=1, device_id=(nbr,),
                                device_id_type=pl.DeviceIdType.MESH)
        pl.semaphore_wait(barrier, 2)

        cp = pltpu.make_async_copy(x_ref, acc_vm, local_sems.at[0])
        cp.start()
        cp.wait()
        x_bf[...] = acc_vm[...].astype(jnp.bfloat16)

        def ag_rdma(h):
            return pltpu.make_async_remote_copy(
                src_ref=x_bf if h == 0 else x_all.at[h],
                dst_ref=x_all.at[h + 1],
                send_sem=ag_send_sems.at[h],
                recv_sem=ag_recv_sems.at[h],
                device_id=(right,),
                device_id_type=pl.DeviceIdType.MESH,
            )

        def rs_rdma(s):
            return pltpu.make_async_remote_copy(
                src_ref=rs_send_vm,
                dst_ref=rs_recv.at[s],
                send_sem=rs_send_sems.at[s],
                recv_sem=rs_recv_sems.at[s],
                device_id=(right,),
                device_id_type=pl.DeviceIdType.MESH,
            )

        def w_dma(t, slot):
            c1 = pltpu.make_async_copy(
                w1_ref.at[:, pl.ds(t * F_T, F_T)], w1_vm.at[slot],
                w1_sems.at[slot])
            c2 = pltpu.make_async_copy(
                w2_ref.at[pl.ds(t * F_T, F_T), :], w2_vm.at[slot],
                w2_sems.at[slot])
            return c1, c2

        def compute_tile(t, slot):
            c1, c2 = w_dma(t, slot)
            c1.wait()
            c2.wait()
            h = jnp.dot(x_bf[...], w1_vm[slot].astype(jnp.bfloat16),
                        preferred_element_type=jnp.float32)
            s = h * jax.nn.sigmoid(h)
            acc_vm[...] += jnp.dot(s.astype(jnp.bfloat16),
                                   w2_vm[slot].astype(jnp.bfloat16),
                                   preferred_element_type=jnp.float32)

        def compute_chunk():
            acc_vm[...] = jnp.zeros_like(acc_vm)
            c1, c2 = w_dma(0, 0)
            c1.start()
            c2.start()

            def pair_body(p, carry):
                t0 = 2 * p
                c1, c2 = w_dma(t0 + 1, 1)
                c1.start()
                c2.start()
                compute_tile(t0, 0)

                @pl.when(t0 + 2 < N_FT)
                def _():
                    c1, c2 = w_dma(t0 + 2, 0)
                    c1.start()
                    c2.start()

                compute_tile(t0 + 1, 1)
                return carry

            lax.fori_loop(0, N_FT // 2, pair_body, 0)

        def load_chunk(k):
            cp = pltpu.make_async_copy(x_all.at[k], x_bf, local_sems.at[0])
            cp.start()
            cp.wait()

        def load_rs(s):
            cp = pltpu.make_async_copy(rs_recv.at[s], rtmp_vm, local_sems.at[0])
            cp.start()
            cp.wait()

        ag0 = ag_rdma(0)
        ag0.start()
        compute_chunk()
        st = pltpu.make_async_copy(acc_vm, p0_buf, local_sems.at[1])
        st.start()
        st.wait()
        ag0.wait()

        ag1 = ag_rdma(1)
        ag1.start()
        load_chunk(1)
        compute_chunk()
        rs_send_vm[...] = acc_vm[...].astype(jnp.bfloat16)
        rs0 = rs_rdma(0)
        rs0.start()
        ag1.wait()

        ag2 = ag_rdma(2)
        ag2.start()
        load_chunk(2)
        compute_chunk()
        rs0.wait()
        load_rs(0)
        rs_send_vm[...] = (acc_vm[...]
                           + rtmp_vm[...].astype(jnp.float32)
                           ).astype(jnp.bfloat16)
        rs1 = rs_rdma(1)
        rs1.start()
        ag2.wait()

        load_chunk(3)
        compute_chunk()
        rs1.wait()
        load_rs(1)
        rs_send_vm[...] = (acc_vm[...]
                           + rtmp_vm[...].astype(jnp.float32)
                           ).astype(jnp.bfloat16)
        rs2 = rs_rdma(2)
        rs2.start()

        ld = pltpu.make_async_copy(p0_buf, acc_vm, local_sems.at[1])
        ld.start()
        ld.wait()
        rs2.wait()
        load_rs(2)
        acc_vm[...] += rtmp_vm[...].astype(jnp.float32)
        st = pltpu.make_async_copy(acc_vm, out_ref, local_sems.at[1])
        st.start()
        st.wait()

    out, _, _, _ = pl.pallas_call(
        body,
        out_shape=[
            jax.ShapeDtypeStruct((M_PER, K), jnp.float32),
            jax.ShapeDtypeStruct((N_DEV, M_PER, K), jnp.bfloat16),
            jax.ShapeDtypeStruct((N_DEV - 1, M_PER, K), jnp.bfloat16),
            jax.ShapeDtypeStruct((M_PER, K), jnp.float32),
        ],
        in_specs=[
            pl.BlockSpec(memory_space=pl.ANY),
            pl.BlockSpec(memory_space=pl.ANY),
            pl.BlockSpec(memory_space=pl.ANY),
        ],
        out_specs=[
            pl.BlockSpec(memory_space=pl.ANY),
            pl.BlockSpec(memory_space=pl.ANY),
            pl.BlockSpec(memory_space=pl.ANY),
            pl.BlockSpec(memory_space=pl.ANY),
        ],
        scratch_shapes=[
            pltpu.VMEM((M_PER, K), jnp.bfloat16),
            pltpu.VMEM((M_PER, K), jnp.float32),
            pltpu.VMEM((2, K, F_T), jnp.float32),
            pltpu.VMEM((2, F_T, K), jnp.float32),
            pltpu.VMEM((M_PER, K), jnp.bfloat16),
            pltpu.VMEM((M_PER, K), jnp.bfloat16),
            pltpu.SemaphoreType.DMA((N_DEV - 1,)),
            pltpu.SemaphoreType.DMA((N_DEV - 1,)),
            pltpu.SemaphoreType.DMA((N_DEV - 1,)),
            pltpu.SemaphoreType.DMA((N_DEV - 1,)),
            pltpu.SemaphoreType.DMA((2,)),
            pltpu.SemaphoreType.DMA((2,)),
            pltpu.SemaphoreType.DMA((2,)),
        ],
        compiler_params=pltpu.CompilerParams(
            collective_id=0,
            vmem_limit_bytes=64 * 1024 * 1024,
        ),
    )(x, W1, W2)
    return out


# device time: 784964 ns/iter; 2.5958x vs baseline; 1.0196x over previous
import jax
import jax.numpy as jnp
from jax import lax
from jax.experimental import pallas as pl
from jax.experimental.pallas import tpu as pltpu

N_DEV = 4
M_PER = 2048
K = 2048
F_LOC = 8192
F_T = 512
N_FT = F_LOC // F_T


def kernel(x, W1, W2):
    def body(x_ref, w1_ref, w2_ref,
             out_ref, x_all, rs_recv, p0_buf,
             x_bf, acc_vm, w1_vm, w2_vm, rs_send_vm, rtmp_vm,
             ag_send_sems, ag_recv_sems, rs_send_sems, rs_recv_sems,
             w1_sems, w2_sems, local_sems):
        my = lax.axis_index("i")
        right = lax.rem(my + 1, N_DEV)
        left = lax.rem(my + N_DEV - 1, N_DEV)

        barrier = pltpu.get_barrier_semaphore()
        for nbr in (left, right):
            pl.semaphore_signal(barrier, inc=1, device_id=(nbr,),
                                device_id_type=pl.DeviceIdType.MESH)
        pl.semaphore_wait(barrier, 2)

        cp = pltpu.make_async_copy(x_ref, acc_vm, local_sems.at[0])
        cp.start()
        cp.wait()
        x_bf[...] = acc_vm[...].astype(jnp.bfloat16)

        def ag_rdma(h):
            return pltpu.make_async_remote_copy(
                src_ref=x_bf if h == 0 else x_all.at[h],
                dst_ref=x_all.at[h + 1],
                send_sem=ag_send_sems.at[h],
                recv_sem=ag_recv_sems.at[h],
                device_id=(right,),
                device_id_type=pl.DeviceIdType.MESH,
            )

        def rs_rdma(s):
            return pltpu.make_async_remote_copy(
                src_ref=rs_send_vm,
                dst_ref=rs_recv.at[s],
                send_sem=rs_send_sems.at[s],
                recv_sem=rs_recv_sems.at[s],
                device_id=(right,),
                device_id_type=pl.DeviceIdType.MESH,
            )

        def w_dma(t, slot):
            c1 = pltpu.make_async_copy(
                w1_ref.at[:, pl.ds(t * F_T, F_T)], w1_vm.at[slot],
                w1_sems.at[slot])
            c2 = pltpu.make_async_copy(
                w2_ref.at[pl.ds(t * F_T, F_T), :], w2_vm.at[slot],
                w2_sems.at[slot])
            return c1, c2

        def compute_tile(t, slot):
            c1, c2 = w_dma(t, slot)
            c1.wait()
            c2.wait()
            h = jnp.dot(x_bf[...], w1_vm[slot].astype(jnp.bfloat16),
                        preferred_element_type=jnp.float32
                        ).astype(jnp.bfloat16)
            s = h * jax.nn.sigmoid(h)
            acc_vm[...] += jnp.dot(s, w2_vm[slot].astype(jnp.bfloat16),
                                   preferred_element_type=jnp.float32)

        def compute_chunk():
            acc_vm[...] = jnp.zeros_like(acc_vm)
            c1, c2 = w_dma(0, 0)
            c1.start()
            c2.start()

            def pair_body(p, carry):
                t0 = 2 * p
                c1, c2 = w_dma(t0 + 1, 1)
                c1.start()
                c2.start()
                compute_tile(t0, 0)

                @pl.when(t0 + 2 < N_FT)
                def _():
                    c1, c2 = w_dma(t0 + 2, 0)
                    c1.start()
                    c2.start()

                compute_tile(t0 + 1, 1)
                return carry

            lax.fori_loop(0, N_FT // 2, pair_body, 0)

        def load_chunk(k):
            cp = pltpu.make_async_copy(x_all.at[k], x_bf, local_sems.at[0])
            cp.start()
            cp.wait()

        def load_rs(s):
            cp = pltpu.make_async_copy(rs_recv.at[s], rtmp_vm, local_sems.at[0])
            cp.start()
            cp.wait()

        ag0 = ag_rdma(0)
        ag0.start()
        compute_chunk()
        st = pltpu.make_async_copy(acc_vm, p0_buf, local_sems.at[1])
        st.start()
        st.wait()
        ag0.wait()

        ag1 = ag_rdma(1)
        ag1.start()
        load_chunk(1)
        compute_chunk()
        rs_send_vm[...] = acc_vm[...].astype(jnp.bfloat16)
        rs0 = rs_rdma(0)
        rs0.start()
        ag1.wait()

        ag2 = ag_rdma(2)
        ag2.start()
        load_chunk(2)
        compute_chunk()
        rs0.wait()
        load_rs(0)
        rs_send_vm[...] = (acc_vm[...]
                           + rtmp_vm[...].astype(jnp.float32)
                           ).astype(jnp.bfloat16)
        rs1 = rs_rdma(1)
        rs1.start()
        ag2.wait()

        load_chunk(3)
        compute_chunk()
        rs1.wait()
        load_rs(1)
        rs_send_vm[...] = (acc_vm[...]
                           + rtmp_vm[...].astype(jnp.float32)
                           ).astype(jnp.bfloat16)
        rs2 = rs_rdma(2)
        rs2.start()

        ld = pltpu.make_async_copy(p0_buf, acc_vm, local_sems.at[1])
        ld.start()
        ld.wait()
        rs2.wait()
        load_rs(2)
        acc_vm[...] += rtmp_vm[...].astype(jnp.float32)
        st = pltpu.make_async_copy(acc_vm, out_ref, local_sems.at[1])
        st.start()
        st.wait()

    out, _, _, _ = pl.pallas_call(
        body,
        out_shape=[
            jax.ShapeDtypeStruct((M_PER, K), jnp.float32),
            jax.ShapeDtypeStruct((N_DEV, M_PER, K), jnp.bfloat16),
            jax.ShapeDtypeStruct((N_DEV - 1, M_PER, K), jnp.bfloat16),
            jax.ShapeDtypeStruct((M_PER, K), jnp.float32),
        ],
        in_specs=[
            pl.BlockSpec(memory_space=pl.ANY),
            pl.BlockSpec(memory_space=pl.ANY),
            pl.BlockSpec(memory_space=pl.ANY),
        ],
        out_specs=[
            pl.BlockSpec(memory_space=pl.ANY),
            pl.BlockSpec(memory_space=pl.ANY),
            pl.BlockSpec(memory_space=pl.ANY),
            pl.BlockSpec(memory_space=pl.ANY),
        ],
        scratch_shapes=[
            pltpu.VMEM((M_PER, K), jnp.bfloat16),
            pltpu.VMEM((M_PER, K), jnp.float32),
            pltpu.VMEM((2, K, F_T), jnp.float32),
            pltpu.VMEM((2, F_T, K), jnp.float32),
            pltpu.VMEM((M_PER, K), jnp.bfloat16),
            pltpu.VMEM((M_PER, K), jnp.bfloat16),
            pltpu.SemaphoreType.DMA((N_DEV - 1,)),
            pltpu.SemaphoreType.DMA((N_DEV - 1,)),
            pltpu.SemaphoreType.DMA((N_DEV - 1,)),
            pltpu.SemaphoreType.DMA((N_DEV - 1,)),
            pltpu.SemaphoreType.DMA((2,)),
            pltpu.SemaphoreType.DMA((2,)),
            pltpu.SemaphoreType.DMA((2,)),
        ],
        compiler_params=pltpu.CompilerParams(
            collective_id=0,
            vmem_limit_bytes=64 * 1024 * 1024,
        ),
    )(x, W1, W2)
    return out
